# baseline (speedup 1.0000x reference)
import functools

import jax
import jax.numpy as jnp
from jax import lax
from jax.experimental import pallas as pl
from jax.experimental.pallas import tpu as pltpu
from jax.experimental.pallas import tpu_sc as plsc

B = 16384
PEP_LEN = 50
MHC_LEN = 34
EMB = 128
PEPTIDE_PAD = 3
MASK_LEN = PEP_LEN - 2 * PEPTIDE_PAD

_info = plsc.get_sparse_core_info()
_NC = _info.num_cores
_NS = _info.num_subcores
_NW = _NC * _NS

_RW = B // _NW
_NBUF = 8
_NGRP = _RW // _NBUF
_PEP_STRIDE = 56
_MHC_STRIDE = 40

_mesh = plsc.VectorSubcoreMesh(core_axis_name="c", subcore_axis_name="s")


@functools.partial(
    pl.kernel,
    mesh=_mesh,
    out_type=[
        jax.ShapeDtypeStruct((B, PEP_LEN, EMB), jnp.float32),
        jax.ShapeDtypeStruct((B, MHC_LEN, EMB), jnp.float32),
    ],
    scratch_types=[
        pltpu.VMEM((_RW * _PEP_STRIDE,), jnp.int32),
        pltpu.VMEM((_RW * _MHC_STRIDE,), jnp.int32),
        pltpu.VMEM((_NBUF, PEP_LEN, EMB), jnp.float32),
    ] + [pltpu.SemaphoreType.DMA] * (2 * _NBUF),
    compiler_params=pltpu.CompilerParams(use_tc_tiling_on_sc=True),
)
def _sc_gather(pep_w, mhc_w, pep_x, mhc_x, pep_out, mhc_out,
               pep_idx_v, mhc_idx_v, rows_v, *sems):
    gsem = sems[:_NBUF]
    wsem = sems[_NBUF:]
    wid = lax.axis_index("s") * _NC + lax.axis_index("c")
    r0 = wid * _RW

    pltpu.sync_copy(
        pep_x.at[pl.ds(r0 * _PEP_STRIDE, _RW * _PEP_STRIDE)], pep_idx_v)
    pltpu.sync_copy(
        mhc_x.at[pl.ds(r0 * _MHC_STRIDE, _RW * _MHC_STRIDE)], mhc_idx_v)

    def run(table, idx_v, out_hbm, seq_len, stride):
        def gd(k, b):
            return pltpu.make_async_copy(
                table.at[idx_v.at[pl.ds(k * stride, seq_len)]],
                rows_v.at[b, pl.ds(0, seq_len), :], gsem[b])

        def wd(k, b):
            return pltpu.make_async_copy(
                rows_v.at[b, pl.ds(0, seq_len), :],
                out_hbm.at[r0 + k], wsem[b])

        for b in range(_NBUF):
            gd(b, b).start()

        def body(g, carry):
            for b in range(_NBUF):
                k = g * _NBUF + b
                gd(k, b).wait()
                wd(k, b).start()
            for b in range(_NBUF):
                k = g * _NBUF + b
                wd(k, b).wait()

                @pl.when(g + 1 < _NGRP)
                def _():
                    gd(k + _NBUF, b).start()
            return carry

        lax.fori_loop(0, _NGRP, body, 0)

    run(pep_w, pep_idx_v, pep_out, PEP_LEN, _PEP_STRIDE)
    run(mhc_w, mhc_idx_v, mhc_out, MHC_LEN, _MHC_STRIDE)


_MASK_RB = 1024


def _mask_body(x_ref, o_ref):
    o_ref[...] = (x_ref[...] != 0).astype(jnp.int32)


_mask_call = pl.pallas_call(
    _mask_body,
    grid=(B // _MASK_RB,),
    in_specs=[pl.BlockSpec((_MASK_RB, MASK_LEN), lambda i: (i, 0))],
    out_specs=pl.BlockSpec((_MASK_RB, MASK_LEN), lambda i: (i, 0)),
    out_shape=jax.ShapeDtypeStruct((B, MASK_LEN), jnp.int32),
)


def kernel(peptide_x, mhc_x, peptide_emb_w, mhc_emb_w):
    pep_x = peptide_x.astype(jnp.int32)
    mhc_x = mhc_x.astype(jnp.int32)
    pep_pad = jnp.pad(pep_x, ((0, 0), (0, _PEP_STRIDE - PEP_LEN)))
    mhc_pad = jnp.pad(mhc_x, ((0, 0), (0, _MHC_STRIDE - MHC_LEN)))
    pep_emb, mhc_emb = _sc_gather(
        peptide_emb_w, mhc_emb_w,
        pep_pad.reshape(B * _PEP_STRIDE), mhc_pad.reshape(B * _MHC_STRIDE))
    mask_in = pep_x[:, PEPTIDE_PAD:PEP_LEN - PEPTIDE_PAD]
    masks = _mask_call(mask_in).astype(bool)
    return (pep_emb, mhc_emb, masks)


# speedup vs baseline: 1.0558x; 1.0558x over previous
import functools

import jax
import jax.numpy as jnp
from jax import lax
from jax.experimental import pallas as pl
from jax.experimental.pallas import tpu as pltpu
from jax.experimental.pallas import tpu_sc as plsc

B = 16384
PEP_LEN = 50
MHC_LEN = 34
EMB = 128
PEPTIDE_PAD = 3
MASK_LEN = PEP_LEN - 2 * PEPTIDE_PAD

_info = plsc.get_sparse_core_info()
_NC = _info.num_cores
_NS = _info.num_subcores
_NW = _NC * _NS

_RW = B // _NW
_NBUF = 8
_NGRP = _RW // _NBUF

_mesh = plsc.VectorSubcoreMesh(core_axis_name="c", subcore_axis_name="s")


def _make_gather(seq_len):
    @functools.partial(
        pl.kernel,
        mesh=_mesh,
        out_type=jax.ShapeDtypeStruct((B, seq_len, EMB), jnp.float32),
        scratch_types=[
            pltpu.VMEM((_RW, seq_len), jnp.int32),
            pltpu.VMEM((_NBUF, seq_len, EMB), jnp.float32),
        ] + [pltpu.SemaphoreType.DMA] * (2 * _NBUF),
    )
    def gather(table, x, out, idx_v, rows_v, *sems):
        gsem = sems[:_NBUF]
        wsem = sems[_NBUF:]
        wid = lax.axis_index("s") * _NC + lax.axis_index("c")
        r0 = wid * _RW

        pltpu.sync_copy(x.at[pl.ds(r0, _RW)], idx_v)

        def gd(k, b):
            return pltpu.make_async_copy(
                table.at[idx_v.at[k]], rows_v.at[b], gsem[b])

        def wd(k, b):
            return pltpu.make_async_copy(
                rows_v.at[b], out.at[r0 + k], wsem[b])

        for b in range(_NBUF):
            gd(b, b).start()

        def body(g, carry):
            for b in range(_NBUF):
                k = g * _NBUF + b
                gd(k, b).wait()
                wd(k, b).start()
            for b in range(_NBUF):
                k = g * _NBUF + b
                wd(k, b).wait()

                @pl.when(g + 1 < _NGRP)
                def _():
                    gd(k + _NBUF, b).start()
            return carry

        lax.fori_loop(0, _NGRP, body, 0)

    return gather


_gather_pep = _make_gather(PEP_LEN)
_gather_mhc = _make_gather(MHC_LEN)


_MASK_RB = 1024


def _mask_body(x_ref, o_ref):
    o_ref[...] = (x_ref[...] != 0).astype(jnp.int32)


_mask_call = pl.pallas_call(
    _mask_body,
    grid=(B // _MASK_RB,),
    in_specs=[pl.BlockSpec((_MASK_RB, MASK_LEN), lambda i: (i, 0))],
    out_specs=pl.BlockSpec((_MASK_RB, MASK_LEN), lambda i: (i, 0)),
    out_shape=jax.ShapeDtypeStruct((B, MASK_LEN), jnp.int32),
)


def kernel(peptide_x, mhc_x, peptide_emb_w, mhc_emb_w):
    pep_x = peptide_x.astype(jnp.int32)
    mhc_x = mhc_x.astype(jnp.int32)
    pep_emb = _gather_pep(peptide_emb_w, pep_x)
    mhc_emb = _gather_mhc(mhc_emb_w, mhc_x)
    mask_in = pep_x[:, PEPTIDE_PAD:PEP_LEN - PEPTIDE_PAD]
    masks = _mask_call(mask_in).astype(bool)
    return (pep_emb, mhc_emb, masks)


# speedup vs baseline: 1.0576x; 1.0017x over previous
import functools

import jax
import jax.numpy as jnp
from jax import lax
from jax.experimental import pallas as pl
from jax.experimental.pallas import tpu as pltpu
from jax.experimental.pallas import tpu_sc as plsc

B = 16384
PEP_LEN = 50
MHC_LEN = 34
EMB = 128
PEPTIDE_PAD = 3
MASK_LEN = PEP_LEN - 2 * PEPTIDE_PAD

_info = plsc.get_sparse_core_info()
_NC = _info.num_cores
_NS = _info.num_subcores
_NW = _NC * _NS

_RW = B // _NW
_NBUF = 8
_NGRP = _RW // _NBUF

_mesh = plsc.VectorSubcoreMesh(core_axis_name="c", subcore_axis_name="s")


def _make_gather(seq_len):
    @functools.partial(
        pl.kernel,
        mesh=_mesh,
        out_type=jax.ShapeDtypeStruct((B, seq_len, EMB), jnp.float32),
        scratch_types=[
            pltpu.VMEM((_RW, seq_len), jnp.int32),
            pltpu.VMEM((_NBUF, seq_len, EMB), jnp.float32),
        ] + [pltpu.SemaphoreType.DMA] * (2 * _NBUF),
        compiler_params=pltpu.CompilerParams(needs_layout_passes=True),
    )
    def gather(table, x, out, idx_v, rows_v, *sems):
        gsem = sems[:_NBUF]
        wsem = sems[_NBUF:]
        wid = lax.axis_index("s") * _NC + lax.axis_index("c")
        r0 = wid * _RW

        pltpu.sync_copy(x.at[pl.ds(r0, _RW)], idx_v)

        def gd(k, b):
            return pltpu.make_async_copy(
                table.at[idx_v.at[k]], rows_v.at[b], gsem[b])

        def wd(k, b):
            return pltpu.make_async_copy(
                rows_v.at[b], out.at[r0 + k], wsem[b])

        for b in range(_NBUF):
            gd(b, b).start()

        def body(g, carry):
            for b in range(_NBUF):
                k = g * _NBUF + b
                gd(k, b).wait()
                wd(k, b).start()
            for b in range(_NBUF):
                k = g * _NBUF + b
                wd(k, b).wait()

                @pl.when(g + 1 < _NGRP)
                def _():
                    gd(k + _NBUF, b).start()
            return carry

        lax.fori_loop(0, _NGRP, body, 0)

    return gather


_gather_pep = _make_gather(PEP_LEN)
_gather_mhc = _make_gather(MHC_LEN)


_MASK_RB = 1024


def _mask_body(x_ref, o_ref):
    o_ref[...] = (x_ref[...] != 0).astype(jnp.int32)


_mask_call = pl.pallas_call(
    _mask_body,
    grid=(B // _MASK_RB,),
    in_specs=[pl.BlockSpec((_MASK_RB, MASK_LEN), lambda i: (i, 0))],
    out_specs=pl.BlockSpec((_MASK_RB, MASK_LEN), lambda i: (i, 0)),
    out_shape=jax.ShapeDtypeStruct((B, MASK_LEN), jnp.int32),
)


def kernel(peptide_x, mhc_x, peptide_emb_w, mhc_emb_w):
    pep_x = peptide_x.astype(jnp.int32)
    mhc_x = mhc_x.astype(jnp.int32)
    pep_emb = _gather_pep(peptide_emb_w, pep_x)
    mhc_emb = _gather_mhc(mhc_emb_w, mhc_x)
    mask_in = pep_x[:, PEPTIDE_PAD:PEP_LEN - PEPTIDE_PAD]
    masks = _mask_call(mask_in).astype(bool)
    return (pep_emb, mhc_emb, masks)
